# Initial kernel scaffold; baseline (speedup 1.0000x reference)
import jax
import jax.numpy as jnp
from jax.experimental import pallas as pl


def kernel(categorical_inputs, numerical_inputs, tables, W1, b1, W2, b2):
    raise NotImplementedError("write your pallas kernel here")


# speedup vs baseline: 2.0005x; 2.0005x over previous
import functools

import jax
import jax.numpy as jnp
from jax import lax
from jax.experimental import pallas as pl
from jax.experimental.pallas import tpu as pltpu
from jax.experimental.pallas import tpu_sc as plsc

NUM_FIELDS = 26
VOCAB = 100000
EMBED_DIM = 32
BATCH = 16384
NUM_NUMERIC = 13
HIDDEN = 128

NC = 2
NS = 16
NW = NC * NS

ROWS_TOTAL = BATCH * NUM_FIELDS
ROWS_PER_W = ROWS_TOTAL // NW
IDX_MINOR = 128
G_PER_W = ROWS_PER_W // IDX_MINOR
CHUNK_G = 13
CHUNK_ROWS = CHUNK_G * IDX_MINOR
N_CHUNKS = G_PER_W // CHUNK_G

_sc_mesh = plsc.VectorSubcoreMesh(core_axis_name="c", subcore_axis_name="s")


@functools.partial(
    pl.kernel,
    mesh=_sc_mesh,
    out_type=jax.ShapeDtypeStruct((ROWS_TOTAL, EMBED_DIM), jnp.float32),
    scratch_types=[
        pltpu.VMEM((G_PER_W, IDX_MINOR), jnp.int32),
        pltpu.VMEM((CHUNK_ROWS, EMBED_DIM), jnp.float32),
        pltpu.VMEM((CHUNK_ROWS, EMBED_DIM), jnp.float32),
        pltpu.SemaphoreType.DMA,
        pltpu.SemaphoreType.DMA,
    ],
    compiler_params=pltpu.CompilerParams(use_tc_tiling_on_sc=False),
)
def _sc_gather(table_hbm, idx_hbm, out_hbm, idx_v, buf0, buf1, sem0, sem1):
    wid = lax.axis_index("s") * NC + lax.axis_index("c")
    row_base = wid * ROWS_PER_W
    pltpu.sync_copy(idx_hbm.at[pl.ds(wid * G_PER_W, G_PER_W)], idx_v)

    bufs = (buf0, buf1)
    sems = (sem0, sem1)

    def fire(c):
        buf = bufs[c % 2]
        sem = sems[c % 2]
        handles = []
        for j in range(CHUNK_G):
            g = c * CHUNK_G + j
            handles.append(
                pltpu.async_copy(
                    table_hbm.at[idx_v.at[g]],
                    buf.at[pl.ds(j * IDX_MINOR, IDX_MINOR)],
                    sem,
                )
            )
        return handles

    def drain_and_store(c, handles):
        for h in handles:
            h.wait()
        pltpu.sync_copy(
            bufs[c % 2],
            out_hbm.at[pl.ds(row_base + c * CHUNK_ROWS, CHUNK_ROWS)],
        )

    prev = None
    for c in range(N_CHUNKS):
        handles = fire(c)
        if prev is not None:
            drain_and_store(*prev)
        prev = (c, handles)
    drain_and_store(*prev)


def _mlp_body(emb_ref, num_ref, w1e_ref, w1n_ref, b1_ref, w2_ref, b2_ref, out_ref):
    x = jnp.dot(emb_ref[...], w1e_ref[...], preferred_element_type=jnp.float32)
    x = x + jnp.dot(num_ref[...], w1n_ref[...], preferred_element_type=jnp.float32)
    x = jax.nn.relu(x + b1_ref[...])
    y = jnp.dot(x, w2_ref[...], preferred_element_type=jnp.float32) + b2_ref[...]
    out_ref[...] = jax.nn.sigmoid(y)


TILE_B = 1024
EMB_W = NUM_FIELDS * EMBED_DIM


def kernel(categorical_inputs, numerical_inputs, tables, W1, b1, W2, b2):
    offsets = jnp.arange(NUM_FIELDS, dtype=jnp.int32) * VOCAB
    flat_idx = (categorical_inputs + offsets[None, :]).reshape(
        ROWS_TOTAL // IDX_MINOR, IDX_MINOR
    )
    flat_table = tables.reshape(NUM_FIELDS * VOCAB, EMBED_DIM)

    gathered = _sc_gather(flat_table, flat_idx)
    emb = gathered.reshape(BATCH, EMB_W)

    W1e = W1[:EMB_W]
    W1n = W1[EMB_W:]
    b1_2d = b1.reshape(1, HIDDEN)
    b2_2d = b2.reshape(1, 1)

    out = pl.pallas_call(
        _mlp_body,
        grid=(BATCH // TILE_B,),
        in_specs=[
            pl.BlockSpec((TILE_B, EMB_W), lambda i: (i, 0)),
            pl.BlockSpec((TILE_B, NUM_NUMERIC), lambda i: (i, 0)),
            pl.BlockSpec((EMB_W, HIDDEN), lambda i: (0, 0)),
            pl.BlockSpec((NUM_NUMERIC, HIDDEN), lambda i: (0, 0)),
            pl.BlockSpec((1, HIDDEN), lambda i: (0, 0)),
            pl.BlockSpec((HIDDEN, 1), lambda i: (0, 0)),
            pl.BlockSpec((1, 1), lambda i: (0, 0)),
        ],
        out_specs=pl.BlockSpec((TILE_B, 1), lambda i: (i, 0)),
        out_shape=jax.ShapeDtypeStruct((BATCH, 1), jnp.float32),
    )(emb, numerical_inputs, W1e, W1n, b1_2d, W2, b2_2d)
    return out


# speedup vs baseline: 2.7633x; 1.3813x over previous
import functools

import jax
import jax.numpy as jnp
from jax import lax
from jax.experimental import pallas as pl
from jax.experimental.pallas import tpu as pltpu
from jax.experimental.pallas import tpu_sc as plsc

NUM_FIELDS = 26
VOCAB = 100000
EMBED_DIM = 32
BATCH = 16384
NUM_NUMERIC = 13
HIDDEN = 128

NC = 2
NS = 16
NW = NC * NS

LANE = 128
QUARTERS = LANE // EMBED_DIM
NF_G = NUM_FIELDS // 2
TAB_ROWS_G = NF_G * VOCAB // QUARTERS
EMB4_G = NF_G * LANE
STRIP = 8
N_STRIPS = BATCH // STRIP
STRIPS_PER_W = N_STRIPS // NW
IDX_PER_STRIP = NF_G * STRIP
IDX_PAD = 128
NBUF = 2

_sc_mesh = plsc.VectorSubcoreMesh(core_axis_name="c", subcore_axis_name="s")


@functools.partial(
    pl.kernel,
    mesh=_sc_mesh,
    out_type=jax.ShapeDtypeStruct((N_STRIPS, STRIP, EMB4_G), jnp.float32),
    scratch_types=[
        pltpu.VMEM((STRIPS_PER_W, IDX_PAD), jnp.int32),
        pltpu.VMEM((STRIP, EMB4_G), jnp.float32),
        pltpu.VMEM((STRIP, EMB4_G), jnp.float32),
        pltpu.SemaphoreType.DMA,
        pltpu.SemaphoreType.DMA,
    ],
    compiler_params=pltpu.CompilerParams(use_tc_tiling_on_sc=True),
)
def _sc_gather(table_hbm, idx_hbm, out_hbm, idx_v, buf0, buf1, sem0, sem1):
    wid = lax.axis_index("s") * NC + lax.axis_index("c")
    strip_base = wid * STRIPS_PER_W
    pltpu.sync_copy(idx_hbm.at[pl.ds(wid * STRIPS_PER_W, STRIPS_PER_W)], idx_v)

    bufs = (buf0, buf1)
    sems = (sem0, sem1)

    def fire(ls, b):
        for t in range(NF_G):
            pltpu.async_copy(
                table_hbm.at[idx_v.at[ls, pl.ds(t * STRIP, STRIP)]],
                bufs[b].at[:, pl.ds(t * LANE, LANE)],
                sems[b],
            )

    def drain_and_store(ls, b):
        pltpu.make_async_copy(
            out_hbm.at[strip_base + ls], bufs[b], sems[b]
        ).wait()
        pltpu.sync_copy(bufs[b], out_hbm.at[strip_base + ls])

    fire(0, 0)
    fire(1, 1)

    def body(k):
        for b in range(NBUF):
            ls_prev = 2 * k + b
            drain_and_store(ls_prev, b)
            fire(ls_prev + 2, b)

    pl.loop(0, STRIPS_PER_W // 2 - 1)(body)

    drain_and_store(STRIPS_PER_W - 2, 0)
    drain_and_store(STRIPS_PER_W - 1, 1)


def _mlp_body(e4a_ref, e4b_ref, cat_ref, num_ref, w1a_ref, w1b_ref, w1n_ref,
              b1_ref, w2_ref, b2_ref, out_ref):
    tb = e4a_ref.shape[0]
    rm = cat_ref[...] // RCHUNK
    q = (lax.broadcasted_iota(jnp.int32, (tb, EMB4_G), 1) % LANE) // EMBED_DIM
    h = jnp.dot(num_ref[...], w1n_ref[...], preferred_element_type=jnp.float32)
    for g, (e4_ref, w_ref) in enumerate(((e4a_ref, w1a_ref),
                                         (e4b_ref, w1b_ref))):
        rmg = rm[:, g * NF_G:(g + 1) * NF_G]
        rmx = jnp.broadcast_to(
            rmg.reshape(tb, NF_G, 1), (tb, NF_G, LANE)
        ).reshape(tb, EMB4_G)
        x = jnp.where(q == rmx, e4_ref[...], 0.0)
        h = h + jnp.dot(x, w_ref[...], preferred_element_type=jnp.float32)
    h = jax.nn.relu(h + b1_ref[...])
    y = jnp.dot(h, w2_ref[...], preferred_element_type=jnp.float32) + b2_ref[...]
    out_ref[...] = jax.nn.sigmoid(y)


TILE_B = 512
EMB_W = NUM_FIELDS * EMBED_DIM

RCHUNK = VOCAB // QUARTERS


def _reformat_body(in_ref, out_ref):
    x = in_ref[0]
    for r in range(QUARTERS):
        out_ref[:, r * EMBED_DIM:(r + 1) * EMBED_DIM] = jnp.transpose(
            x[:, r * RCHUNK:(r + 1) * RCHUNK], (1, 0)
        )


def _reformat_table(tables_t, g):
    return pl.pallas_call(
        _reformat_body,
        grid=(NF_G,),
        in_specs=[
            pl.BlockSpec((1, EMBED_DIM, VOCAB), lambda f: (g * NF_G + f, 0, 0)),
        ],
        out_specs=pl.BlockSpec((RCHUNK, LANE), lambda f: (f, 0)),
        out_shape=jax.ShapeDtypeStruct((TAB_ROWS_G, LANE), jnp.float32),
        compiler_params=pltpu.CompilerParams(
            vmem_limit_bytes=110 * 1024 * 1024,
        ),
    )(tables_t)


def kernel(categorical_inputs, numerical_inputs, tables, W1, b1, W2, b2):
    tables_t = jnp.transpose(tables, (0, 2, 1))
    offsets = jnp.arange(NF_G, dtype=jnp.int32) * RCHUNK

    emb4s = []
    for g in range(2):
        cat_g = categorical_inputs[:, g * NF_G:(g + 1) * NF_G]
        flat4 = cat_g % RCHUNK + offsets[None, :]
        idx_perm = jnp.transpose(
            flat4.reshape(N_STRIPS, STRIP, NF_G), (0, 2, 1)
        ).reshape(N_STRIPS, IDX_PER_STRIP)
        idx_perm = jnp.pad(idx_perm, ((0, 0), (0, IDX_PAD - IDX_PER_STRIP)))
        table4 = _reformat_table(tables_t, g)
        emb4s.append(_sc_gather(table4, idx_perm).reshape(BATCH, EMB4_G))

    W1e = W1[:EMB_W].reshape(NUM_FIELDS, 1, EMBED_DIM, HIDDEN)
    W1x = jnp.broadcast_to(
        W1e, (NUM_FIELDS, QUARTERS, EMBED_DIM, HIDDEN)
    ).reshape(NUM_FIELDS * LANE, HIDDEN)
    W1n = W1[EMB_W:]
    b1_2d = b1.reshape(1, HIDDEN)
    b2_2d = b2.reshape(1, 1)

    out = pl.pallas_call(
        _mlp_body,
        grid=(BATCH // TILE_B,),
        in_specs=[
            pl.BlockSpec((TILE_B, EMB4_G), lambda i: (i, 0)),
            pl.BlockSpec((TILE_B, EMB4_G), lambda i: (i, 0)),
            pl.BlockSpec((TILE_B, NUM_FIELDS), lambda i: (i, 0)),
            pl.BlockSpec((TILE_B, NUM_NUMERIC), lambda i: (i, 0)),
            pl.BlockSpec((EMB4_G, HIDDEN), lambda i: (0, 0)),
            pl.BlockSpec((EMB4_G, HIDDEN), lambda i: (0, 0)),
            pl.BlockSpec((NUM_NUMERIC, HIDDEN), lambda i: (0, 0)),
            pl.BlockSpec((1, HIDDEN), lambda i: (0, 0)),
            pl.BlockSpec((HIDDEN, 1), lambda i: (0, 0)),
            pl.BlockSpec((1, 1), lambda i: (0, 0)),
        ],
        out_specs=pl.BlockSpec((TILE_B, 1), lambda i: (i, 0)),
        out_shape=jax.ShapeDtypeStruct((BATCH, 1), jnp.float32),
    )(emb4s[0], emb4s[1], categorical_inputs, numerical_inputs,
      W1x[:EMB4_G], W1x[EMB4_G:], W1n, b1_2d, W2, b2_2d)
    return out
